# baseline (speedup 1.0000x reference)
import functools

import jax
import jax.numpy as jnp
from jax import lax
from jax.experimental import pallas as pl
from jax.experimental.pallas import tpu as pltpu
from jax.experimental.pallas import tpu_sc as plsc

B = 4096
S = 200
D = 64
NW = 32
BW = B // NW
ROWS = B * S
SCALE = 8.0


def _mesh():
    return plsc.VectorSubcoreMesh(core_axis_name="c", subcore_axis_name="s")


@functools.partial(
    pl.kernel,
    mesh=_mesh(),
    out_type=jax.ShapeDtypeStruct((S, D, B), jnp.float32),
    compiler_params=pltpu.CompilerParams(
        use_tc_tiling_on_sc=False, needs_layout_passes=False),
    scratch_types=[
        pltpu.VMEM((2, BW), jnp.int32),
        pltpu.VMEM((2, BW), jnp.int32),
        pltpu.VMEM((2, BW), jnp.int32),
        pltpu.VMEM((2, BW, 2 * D), jnp.float32),
        pltpu.VMEM((2, D, BW), jnp.float32),
        pltpu.VMEM((S * D,), jnp.float32),
        pltpu.SemaphoreType.DMA,
        pltpu.SemaphoreType.DMA,
        pltpu.SemaphoreType.DMA,
        pltpu.SemaphoreType.DMA,
        pltpu.SemaphoreType.DMA,
        pltpu.SemaphoreType.DMA,
    ],
)
def _embed(idx_hbm, tok_hbm, pos_hbm, out_hbm, idx_v, q_v, p_v, g_v, t_v,
           pos_v, isem0, isem1, gsem0, gsem1, osem0, osem1):
    isem = (isem0, isem1)
    gsem = (gsem0, gsem1)
    osem = (osem0, osem1)
    wid = lax.axis_index("s") * 2 + lax.axis_index("c")
    b0 = wid * BW
    iota = lax.iota(jnp.int32, 16)
    pltpu.sync_copy(pos_hbm, pos_v)

    def idx_copy(s, b):
        return pltpu.make_async_copy(
            idx_hbm.at[pl.ds(s * B + b0, BW)], idx_v.at[b], isem[b])

    def gather(b):
        return pltpu.make_async_copy(tok_hbm.at[q_v.at[b]], g_v.at[b],
                                     gsem[b])

    def out_copy(s, b):
        return pltpu.make_async_copy(
            t_v.at[b], out_hbm.at[s, pl.ds(0, D), pl.ds(b0, BW)], osem[b])

    def derive(b):
        for i in range(BW // 16):
            sl = pl.ds(i * 16, 16)
            raw = idx_v[b, sl]
            q_v[b, sl] = raw >> 1
            p_v[b, sl] = (raw & 1) * D

    pltpu.sync_copy(idx_hbm.at[pl.ds(b0, BW)], idx_v.at[0])
    derive(0)
    gather(0).start()
    idx_copy(1, 1).start()

    def s_body(s, carry):
        b = lax.rem(s, 2)
        for bs in range(2):
            @pl.when(b == bs)
            def _unit():
                nb = 1 - bs

                @pl.when(s + 1 < S)
                def _prefetch():
                    idx_copy(s + 1, nb).wait()
                    derive(nb)
                    gather(nb).start()

                @pl.when(s + 2 < S)
                def _next_idx():
                    idx_copy(s + 2, bs).start()

                gather(bs).wait()

                @pl.when(s >= 2)
                def _drain_out():
                    out_copy(s, bs).wait()

                def j_body(j, carry2):
                    pos_j = plsc.load_gather(
                        pos_v, [jnp.full((16,), s * D + j, jnp.int32)])
                    for bg in range(BW // 16):
                        sl = pl.ds(bg * 16, 16)
                        rows = iota + bg * 16
                        half = p_v[bs, sl]
                        vals = plsc.load_gather(g_v.at[bs],
                                                [rows, half + j])
                        t_v[bs, j, sl] = vals * SCALE + pos_j
                    return carry2

                lax.fori_loop(0, D, j_body, 0, unroll=2)
                out_copy(s, bs).start()
        return carry

    lax.fori_loop(0, S, s_body, 0)
    for bs in range(2):
        out_copy(S - 2 + bs, bs).wait()


def kernel(inputs, tok_table, pos_table):
    idx = inputs.T.reshape(ROWS)
    tok2 = tok_table.reshape(500000, 2 * D)
    pos = pos_table.reshape(S * D)
    out = _embed(idx, tok2, pos)
    return jnp.transpose(out, (2, 0, 1))


# speedup vs baseline: 2.0410x; 2.0410x over previous
import functools

import jax
import jax.numpy as jnp
from jax import lax
from jax.experimental import pallas as pl
from jax.experimental.pallas import tpu as pltpu
from jax.experimental.pallas import tpu_sc as plsc

B = 4096
S = 200
D = 64
NC = 2
NS = 16
NW = NC * NS
ROWS = B * S
RPW = ROWS // NW
CB = 2
C = CB * S
NCH = RPW // C
SCALE = 8.0


def _mesh():
    return plsc.VectorSubcoreMesh(core_axis_name="c", subcore_axis_name="s")


@functools.partial(
    pl.kernel,
    mesh=_mesh(),
    out_type=jax.ShapeDtypeStruct((B, S, D), jnp.float32),
    compiler_params=pltpu.CompilerParams(use_tc_tiling_on_sc=False),
    scratch_types=[
        pltpu.VMEM((2, C), jnp.int32),
        pltpu.VMEM((2, C, D), jnp.float32),
        pltpu.VMEM((2, CB, S, D), jnp.float32),
        pltpu.VMEM((S, D), jnp.float32),
        pltpu.SemaphoreType.DMA,
        pltpu.SemaphoreType.DMA,
        pltpu.SemaphoreType.DMA,
        pltpu.SemaphoreType.DMA,
        pltpu.SemaphoreType.DMA,
        pltpu.SemaphoreType.DMA,
    ],
)
def _embed(idx_hbm, tok_hbm, pos_hbm, out_hbm, idx_v, g_v, o_v, pos_v,
           isem0, isem1, gsem0, gsem1, osem0, osem1):
    isem = (isem0, isem1)
    gsem = (gsem0, gsem1)
    osem = (osem0, osem1)
    wid = lax.axis_index("s") * NC + lax.axis_index("c")
    base = wid * RPW
    batch0 = wid * (RPW // S)
    pltpu.sync_copy(pos_hbm, pos_v)

    for b in range(2):
        pltpu.sync_copy(idx_hbm.at[pl.ds(base + b * C, C)], idx_v.at[b])
        pltpu.make_async_copy(tok_hbm.at[idx_v.at[b]], g_v.at[b],
                              gsem[b]).start()

    def out_copy(c, b):
        return pltpu.make_async_copy(
            o_v.at[b], out_hbm.at[pl.ds(batch0 + c * CB, CB)], osem[b])

    def outer(gi, carry):
        for b in range(2):
            c = 2 * gi + b
            row0 = base + c * C
            pltpu.make_async_copy(tok_hbm.at[idx_v.at[b]], g_v.at[b],
                                  gsem[b]).wait()
            @pl.when(c < NCH - 2)
            def _stage():
                pltpu.make_async_copy(
                    idx_hbm.at[pl.ds(row0 + 2 * C, C)], idx_v.at[b],
                    isem[b]).start()

            @pl.when(c >= 2)
            def _drain():
                out_copy(c, b).wait()

            def row_body(r, carry2):
                for sb in range(CB):
                    row = sb * S + r
                    for j in range(D // 16):
                        sl = pl.ds(j * 16, 16)
                        o_v[b, sb, r, sl] = (
                            g_v[b, row, sl] * SCALE + pos_v[r, sl])
                return carry2

            lax.fori_loop(0, S, row_body, 0, unroll=2)

            out_copy(c, b).start()

            @pl.when(c < NCH - 2)
            def _next_gather():
                pltpu.make_async_copy(
                    idx_hbm.at[pl.ds(row0 + 2 * C, C)], idx_v.at[b],
                    isem[b]).wait()
                pltpu.make_async_copy(tok_hbm.at[idx_v.at[b]], g_v.at[b],
                                      gsem[b]).start()
        return carry

    lax.fori_loop(0, NCH // 2, outer, 0)
    for b in range(2):
        out_copy(NCH - 2 + b, b).wait()


def kernel(inputs, tok_table, pos_table):
    idx = inputs.reshape(ROWS).astype(jnp.int32)
    return _embed(idx, tok_table, pos_table)
